# Initial kernel scaffold; baseline (speedup 1.0000x reference)
import jax
import jax.numpy as jnp
from jax.experimental import pallas as pl


def kernel(x, w0, params, edge_index, graph_len, prompt_id, scalar):
    raise NotImplementedError("write your pallas kernel here")


# speedup vs baseline: 5.9984x; 5.9984x over previous
import functools

import jax
import jax.numpy as jnp
from jax import lax
from jax.experimental import pallas as pl
from jax.experimental.pallas import tpu as pltpu
from jax.experimental.pallas import tpu_sc as plsc

NUM_CORES = 2
NUM_SUBCORES = 16
NUM_WORKERS = NUM_CORES * NUM_SUBCORES
CHUNK = 128


def _sc_scatter_sum(h, src, dst, zeros_nf):
    n, f = h.shape
    e = src.shape[0]
    ew = e // NUM_WORKERS
    nch = ew // CHUNK
    rem = ew - nch * CHUNK

    mesh = plsc.VectorSubcoreMesh(
        core_axis_name="c", subcore_axis_name="s",
        num_cores=NUM_CORES, num_subcores=NUM_SUBCORES)

    @functools.partial(
        pl.kernel,
        out_type=jax.ShapeDtypeStruct((NUM_CORES, n, f), jnp.float32),
        mesh=mesh,
        scratch_types=[
            pltpu.VMEM((CHUNK,), jnp.int32),
            pltpu.VMEM((CHUNK,), jnp.int32),
            pltpu.VMEM((CHUNK, f), jnp.float32),
            pltpu.VMEM((16,), jnp.int32),
            pltpu.VMEM((16,), jnp.int32),
            pltpu.VMEM_SHARED((n, f), jnp.float32),
            pltpu.SemaphoreType.DMA,
        ],
        compiler_params=pltpu.CompilerParams(use_tc_tiling_on_sc=False),
    )
    def k(h_hbm, src_hbm, dst_hbm, z_hbm, out_hbm,
          sidx, didx, rows, sidx2, didx2, acc, sem):
        cid = lax.axis_index("c")
        sid = lax.axis_index("s")
        wid = cid * NUM_SUBCORES + sid

        @pl.when(sid == 0)
        def _():
            pltpu.sync_copy(z_hbm, acc)
        plsc.subcore_barrier()

        wbase = wid * ew

        def body(i, _):
            base = pl.multiple_of(wbase + i * CHUNK, 8)
            pltpu.sync_copy(src_hbm.at[pl.ds(base, CHUNK)], sidx)
            pltpu.async_copy(h_hbm.at[sidx], rows, sem).wait()
            pltpu.sync_copy(dst_hbm.at[pl.ds(base, CHUNK)], didx)
            pltpu.sync_copy(rows, acc.at[didx], add=True)
            return ()

        lax.fori_loop(0, nch, body, (), unroll=False)

        if rem:
            base = pl.multiple_of(wbase + nch * CHUNK, 8)
            pltpu.sync_copy(src_hbm.at[pl.ds(base, rem)], sidx2)
            pltpu.async_copy(h_hbm.at[sidx2], rows.at[pl.ds(0, rem)], sem).wait()
            pltpu.sync_copy(dst_hbm.at[pl.ds(base, rem)], didx2)
            pltpu.sync_copy(rows.at[pl.ds(0, rem)], acc.at[didx2], add=True)

        plsc.subcore_barrier()

        @pl.when(sid == 0)
        def _():
            pltpu.sync_copy(acc, out_hbm.at[cid])

    return k(h, src, dst, zeros_nf)


def _tc_layer(h, agg, w0, w1, b1, w2, b2, gamma, beta, b_graphs, n_per):
    n, f = h.shape
    hdim = w2.shape[1]

    def body(h_ref, a_ref, w0_ref, w1_ref, b1_ref, w2_ref, b2_ref,
             g_ref, be_ref, out_ref, pool_ref):
        a = a_ref[...]
        rst = h_ref[...] + a[0] + a[1]
        if w0 is not None:
            rst = rst * w0_ref[...]
        z = jnp.dot(rst, w1_ref[...], preferred_element_type=jnp.float32)
        z = jnp.maximum(z + b1_ref[...], 0.0)
        z = jnp.dot(z, w2_ref[...], preferred_element_type=jnp.float32)
        hh = jnp.maximum(z + b2_ref[...], 0.0)
        mean = jnp.mean(hh, axis=0, keepdims=True)
        var = jnp.mean((hh - mean) * (hh - mean), axis=0, keepdims=True)
        hh = (hh - mean) * lax.rsqrt(var + 1e-5) * g_ref[...] + be_ref[...]
        out_ref[...] = hh
        rows = lax.broadcasted_iota(jnp.int32, (b_graphs, n), 1) // n_per
        seg = lax.broadcasted_iota(jnp.int32, (b_graphs, n), 0)
        ind = jnp.where(rows == seg, 1.0, 0.0).astype(jnp.float32)
        pool_ref[...] = jnp.dot(ind, hh, preferred_element_type=jnp.float32)

    args = [h, agg]
    if w0 is None:
        w0_in = jnp.zeros((1, f), jnp.float32)
    else:
        w0_in = w0
    args += [w0_in, w1, b1.reshape(1, -1), w2, b2.reshape(1, -1),
             gamma.reshape(1, -1), beta.reshape(1, -1)]
    return pl.pallas_call(
        body,
        out_shape=[
            jax.ShapeDtypeStruct((n, hdim), jnp.float32),
            jax.ShapeDtypeStruct((b_graphs, hdim), jnp.float32),
        ],
    )(*args)


def _tc_combine(p1, p2, p3, n_per):
    b, hdim = p1.shape

    def body(p1_ref, p2_ref, p3_ref, xcat_ref, emb_ref):
        xcat_ref[:, 0:hdim] = p1_ref[...]
        xcat_ref[:, hdim:2 * hdim] = p2_ref[...]
        xcat_ref[:, 2 * hdim:3 * hdim] = p3_ref[...]
        emb_ref[:, 0:hdim] = p1_ref[...] * (1.0 / n_per)
        emb_ref[:, hdim:2 * hdim] = p2_ref[...] * (1.0 / n_per)
        emb_ref[:, 2 * hdim:3 * hdim] = p3_ref[...] * (1.0 / n_per)

    return pl.pallas_call(
        body,
        out_shape=[
            jax.ShapeDtypeStruct((b, 3 * hdim), jnp.float32),
            jax.ShapeDtypeStruct((b, 3 * hdim), jnp.float32),
        ],
    )(p1, p2, p3)


def kernel(x, w0, params, edge_index, graph_len, prompt_id, scalar):
    n, d = x.shape
    b_graphs = graph_len.shape[0]
    n_per = n // b_graphs
    src = edge_index[0]
    dst = edge_index[1]
    hdim = params[0][2].shape[1]

    zeros_d = jnp.zeros((n, d), jnp.float32)
    zeros_h = jnp.zeros((n, hdim), jnp.float32)

    agg = _sc_scatter_sum(x, src, dst, zeros_d)
    w1, b1, w2, b2, gamma, beta = params[0]
    h, p1 = _tc_layer(x, agg, w0, w1, b1, w2, b2, gamma, beta, b_graphs, n_per)

    w1, b1, w2, b2, gamma, beta = params[1]
    agg = _sc_scatter_sum(h, src, dst, zeros_h)
    h, p2 = _tc_layer(h, agg, None, w1, b1, w2, b2, gamma, beta, b_graphs, n_per)

    w1, b1, w2, b2, gamma, beta = params[2]
    agg = _sc_scatter_sum(h, src, dst, zeros_h)
    h, p3 = _tc_layer(h, agg, None, w1, b1, w2, b2, gamma, beta, b_graphs, n_per)

    xcat, emb = _tc_combine(p1, p2, p3, n_per)
    return (xcat, emb)
